# baseline (speedup 1.0000x reference)
import functools

import jax
import jax.numpy as jnp
from jax import lax
from jax.experimental import pallas as pl
from jax.experimental.pallas import tpu as pltpu
from jax.experimental.pallas import tpu_sc as plsc

_NG = 878
_NG0 = 1154
_ND = 383
_NM = 495
_FEAT = 256
_H = 8
_FA = 64
_HF = _H * _FA
_ZW = _HF + 16
_OUT = 64
_B = 16384

_NC = 2
_NS = 16
_NW = _NC * _NS

_T0 = 880
_T1 = 1160
_N0P = 896
_N1S = 512
_N12 = 896
_R0 = _N0P // _NS
_R12 = _N12 // _NS

_CH = 64
_E0, _E1 = 20000, 10000
_NCH = 10
_EP = _NW * _NCH * _CH

_PW = _B // _NW


def _prep_body(x0, x1, p0, p1, q0, q1, zt0, zt1, esd0, esd1):
    hi = lax.Precision.HIGHEST
    f32 = jnp.float32

    def mm(a, b):
        return lax.dot_general(a, b, (((1,), (0,)), ((), ())),
                               precision=hi, preferred_element_type=f32)

    zt0[...] = jnp.concatenate(
        [mm(x0[...], p0[...]), jnp.zeros((_T0 - _NG, _ZW), f32)], axis=0)
    zt1[...] = jnp.concatenate(
        [mm(x1[...], p1[...]), jnp.zeros((_T1 - _NG0, _ZW), f32)], axis=0)
    e0 = mm(x0[...], q0[...])
    e1 = mm(x1[...], q1[...])
    esd0[...] = jnp.concatenate(
        [jnp.concatenate([e0, e0], axis=1),
         jnp.zeros((_T0 - _NG, 16), f32)], axis=0)
    esd1[...] = jnp.concatenate(
        [jnp.concatenate([e1, e1], axis=1),
         jnp.zeros((_T1 - _NG0, 16), f32)], axis=0)


def _prep_call(x0, x1, p0, p1, q0, q1):
    return pl.pallas_call(
        _prep_body,
        out_shape=[
            jax.ShapeDtypeStruct((_T0, _ZW), jnp.float32),
            jax.ShapeDtypeStruct((_T1, _ZW), jnp.float32),
            jax.ShapeDtypeStruct((_T0, 16), jnp.float32),
            jax.ShapeDtypeStruct((_T1, 16), jnp.float32),
        ],
    )(x0, x1, p0, p1, q0, q1)


@functools.cache
def _mesh():
    return plsc.VectorSubcoreMesh(core_axis_name="c", subcore_axis_name="s",
                                  num_cores=_NC, num_subcores=_NS)


@functools.cache
def _edge_kernel():
    return functools.partial(
        pl.kernel,
        out_type=[
            jax.ShapeDtypeStruct((_NC, _N0P, _ZW), jnp.float32),
            jax.ShapeDtypeStruct((_NC, _N12, _ZW), jnp.float32),
        ],
        mesh=_mesh(),
        compiler_params=pltpu.CompilerParams(needs_layout_passes=False,
                                             use_tc_tiling_on_sc=False),
        scratch_types=[
            pltpu.VMEM_SHARED((_N0P, _ZW), jnp.float32),
            pltpu.VMEM_SHARED((_N12, _ZW), jnp.float32),
            pltpu.VMEM((_CH, _ZW), jnp.float32),
            pltpu.VMEM((_CH, _ZW), jnp.float32),
            pltpu.VMEM((_CH, 16), jnp.float32),
            pltpu.VMEM((_CH, 16), jnp.float32),
            pltpu.VMEM((_NCH, _CH), jnp.int32),
            pltpu.VMEM((_NCH, _CH), jnp.int32),
            pltpu.SemaphoreType.DMA,
            pltpu.SemaphoreType.DMA,
            pltpu.SemaphoreType.DMA,
            pltpu.SemaphoreType.DMA,
            pltpu.SemaphoreType.DMA,
            pltpu.SemaphoreType.DMA,
        ],
    )(_edge_body)


def _edge_body(zt0, esd0, zt1, esd1, s0r, d0r, s12r, d12r, zer,
               out0, out12, acc0, acc12,
               zb0, zb1, eb0, eb1, s2d, d2d,
               gz0, gz1, ge0, ge1, sc0, sc1):
    c = lax.axis_index("c")
    s = lax.axis_index("s")
    wid = c * _NS + s
    zb = (zb0, zb1)
    eb = (eb0, eb1)
    gz = (gz0, gz1)
    ge = (ge0, ge1)
    sc = (sc0, sc1)

    pltpu.sync_copy(zer.at[pl.ds(0, _R0)], acc0.at[pl.ds(s * _R0, _R0)])
    pltpu.sync_copy(zer, acc12.at[pl.ds(s * _R12, _R12)])
    plsc.subcore_barrier()

    iota16 = lax.iota(jnp.int32, 16)
    lane_lt8 = iota16 < 8

    def remap0(v, gidx):
        return jnp.where(v < _NG, v, _NG + 2 + iota16)

    def remap12(v, gidx):
        t = v - _ND
        r1 = jnp.where((t >= 0) & (t < _NM), t, _NM + iota16)
        r2 = jnp.where(v < _ND, v + 511, _NM + iota16)
        return jnp.where(gidx < _E1, r1, r2)

    def do_graph(zt, esd, sarr, darr, acc, remap):
        pltpu.sync_copy(sarr.at[wid], s2d)
        pltpu.sync_copy(darr.at[wid], d2d)

        def issue(j, b):
            pltpu.async_copy(zt.at[s2d.at[j]], zb[b], gz[b])
            pltpu.async_copy(esd.at[d2d.at[j]], eb[b], ge[b])

        def wait_gather(j, b):
            pltpu.make_async_copy(zt.at[s2d.at[j]], zb[b], gz[b]).wait()
            pltpu.make_async_copy(esd.at[d2d.at[j]], eb[b], ge[b]).wait()

        def wait_scatter(j, b):
            pltpu.make_async_copy(zb[b], acc.at[d2d.at[j]], sc[b]).wait()

        issue(0, 0)

        def step(j, b):
            zrows = zb[b]
            edrows = eb[b]

            @pl.when(j >= 1)
            def _():
                wait_scatter(j - 1, 1 - b)

            @pl.when(j < _NCH - 1)
            def _():
                issue(j + 1, 1 - b)

            wait_gather(j, b)

            base = wid * (_NCH * _CH) + j * _CH
            for q in range(_CH // 16):
                sl = pl.ds(q * 16, 16)
                gidx = base + q * 16 + iota16
                d2d[j, sl] = remap(d2d[j, sl], gidx)

            def edge(e, carry2):
                esv = zrows[e, pl.ds(_HF, 16)]
                edv = edrows[e, :]
                x = esv + edv
                lk = jnp.where(x >= 0, x, jnp.float32(0.2) * x)
                w = jnp.exp(lk)
                wm = jnp.where(lane_lt8, w, jnp.float32(0.0))
                dn = lax.GatherDimensionNumbers(
                    offset_dims=(), collapsed_slice_dims=(0,),
                    start_index_map=(0,))
                for h in range(_H):
                    wspl = lax.gather(
                        w, jnp.full((16, 1), h, jnp.int32), dn, (1,),
                        mode=lax.GatherScatterMode.PROMISE_IN_BOUNDS)
                    for q in range(4):
                        t = h * 4 + q
                        sl = pl.ds(t * 16, 16)
                        zrows[e, sl] = zrows[e, sl] * wspl
                zrows[e, pl.ds(_HF, 16)] = wm
                return carry2

            lax.fori_loop(0, _CH, edge, 0)
            pltpu.async_copy(zrows, acc.at[d2d.at[j]], sc[b], add=True)

        def pair(j2, carry):
            step(j2 * 2, 0)
            step(j2 * 2 + 1, 1)
            return carry

        lax.fori_loop(0, _NCH // 2, pair, 0)
        wait_scatter(_NCH - 1, 1)

    do_graph(zt0, esd0, s0r, d0r, acc0, remap0)
    do_graph(zt1, esd1, s12r, d12r, acc12, remap12)
    plsc.subcore_barrier()

    pltpu.sync_copy(acc0.at[pl.ds(s * _R0, _R0)],
                    out0.at[c, pl.ds(s * _R0, _R0)])
    pltpu.sync_copy(acc12.at[pl.ds(s * _R12, _R12)],
                    out12.at[c, pl.ds(s * _R12, _R12)])


def _fuse_body(p0, p12, dsim, msim, dfcw, dfcb,
               mfcw, mfcb, hfcw, hfcb, h_ref):
    hi = lax.Precision.HIGHEST
    f32 = jnp.float32

    def mm(a, b):
        return lax.dot_general(a, b, (((1,), (0,)), ((), ())),
                               precision=hi, preferred_element_type=f32)

    def elu(x):
        return jnp.where(x > 0, x, jnp.exp(jnp.minimum(x, 0.0)) - 1.0)

    ii = lax.broadcasted_iota(jnp.int32, (_H, _HF), 0)
    jj = lax.broadcasted_iota(jnp.int32, (_H, _HF), 1)
    r8 = (jj // _FA == ii).astype(f32)

    def hpart(p):
        a = p[0] + p[1]
        msg = a[:, :_HF]
        den = a[:, _HF:_HF + _H]
        dinv = 1.0 / (den + 1e-9)
        return elu(msg * mm(dinv, r8))

    h0 = hpart(p0[...])
    h12 = hpart(p12[...])
    hs1 = 0.5 * (h0[:_ND + 1] + h12[511:511 + _ND + 1])
    hs2 = 0.5 * (h0[_ND:_ND + _NM + 1] + h12[:_NM + 1])
    dss = jnp.concatenate(
        [dsim[:_ND], jnp.zeros((1, _ND), f32)], axis=0)
    mss = jnp.concatenate(
        [msim[_ND:_NG], jnp.zeros((1, _NM), f32)], axis=0)
    dw = dfcw[...]
    mw = mfcw[...]
    hd = elu(mm(hs1, dw[:_HF]) + mm(dss, dw[_HF:])
             + dfcb[...].reshape(1, _OUT))
    hm = elu(mm(hs2, mw[:_HF]) + mm(mss, mw[_HF:])
             + mfcb[...].reshape(1, _OUT))
    hcat = jnp.concatenate([hd[:_ND], hm[:_NM]], axis=0)
    h = elu(mm(hcat, hfcw[...]) + hfcb[...].reshape(1, _OUT))
    h_ref[...] = jnp.concatenate(
        [h, jnp.zeros((_N0P - _NG, _OUT), f32)], axis=0)


def _fuse_call(p0, p12, dsim, msim, dfcw, dfcb, mfcw, mfcb, hfcw, hfcb):
    return pl.pallas_call(
        _fuse_body,
        out_shape=jax.ShapeDtypeStruct((_N0P, _OUT), jnp.float32),
    )(p0, p12, dsim, msim, dfcw, dfcb, mfcw, mfcb, hfcw, hfcb)


@functools.cache
def _pair_kernel():
    return functools.partial(
        pl.kernel,
        out_type=jax.ShapeDtypeStruct((_B,), jnp.float32),
        mesh=_mesh(),
        compiler_params=pltpu.CompilerParams(needs_layout_passes=False, use_tc_tiling_on_sc=False),
        scratch_types=[
            pltpu.VMEM((_N0P * _OUT,), jnp.float32),
            pltpu.VMEM((_PW,), jnp.int32),
            pltpu.VMEM((_PW,), jnp.int32),
            pltpu.VMEM((_PW,), jnp.float32),
        ],
    )(_pair_body)


def _pair_body(hflat, dis, mir, out, hbuf, dbuf, mbuf, obuf):
    c = lax.axis_index("c")
    s = lax.axis_index("s")
    wid = c * _NS + s
    pltpu.sync_copy(hflat, hbuf)
    pltpu.sync_copy(dis.at[pl.ds(wid * _PW, _PW)], dbuf)
    pltpu.sync_copy(mir.at[pl.ds(wid * _PW, _PW)], mbuf)

    def grp(g, carry):
        dv = dbuf[pl.ds(g * 16, 16)] * _OUT
        mv = mbuf[pl.ds(g * 16, 16)] * _OUT
        accs = [jnp.zeros((16,), jnp.float32) for _ in range(4)]
        for k in range(_OUT):
            a = plsc.load_gather(hbuf, [dv + k])
            b = plsc.load_gather(hbuf, [mv + k])
            accs[k % 4] = accs[k % 4] + a * b
        acc = (accs[0] + accs[1]) + (accs[2] + accs[3])
        obuf[pl.ds(g * 16, 16)] = 1.0 / (1.0 + jnp.exp(-acc))
        return carry

    lax.fori_loop(0, _PW // 16, grp, 0)
    pltpu.sync_copy(obuf, out.at[pl.ds(wid * _PW, _PW)])


def kernel(x_g, x_g0, d_sim, m_sim, edge_index_md, edge_index_ml,
           edge_index_dl, diseases, mirnas, gat_W, gat_asrc, gat_adst,
           mp_W, mp_asrc, mp_adst, sem_P1, sem_b1, sem_P2, mfc_W, mfc_b,
           dfc_W, dfc_b, hfc_W, hfc_b):
    f32 = jnp.float32
    i32 = jnp.int32

    wf0 = gat_W.reshape(_FEAT, _HF)
    was0 = jnp.einsum('fhk,hk->fh', gat_W, gat_asrc)
    wad0 = jnp.einsum('fhk,hk->fh', gat_W, gat_adst)
    p0w = jnp.concatenate([wf0, was0, wad0], axis=1)
    wf1 = jnp.transpose(mp_W, (1, 0, 2)).reshape(_FEAT, _HF)
    was1 = jnp.einsum('hfk,hk->fh', mp_W, mp_asrc)
    wad1 = jnp.einsum('hfk,hk->fh', mp_W, mp_adst)
    p1w = jnp.concatenate([wf1, was1, wad1], axis=1)

    zt0, zt1, esd0, esd1 = _prep_call(x_g, x_g0, p0w, p1w, wad0, wad1)

    npad0 = _EP - _E0
    s0 = jnp.concatenate([edge_index_md[0].astype(i32),
                          jnp.zeros((npad0,), i32)]).reshape(_NW, _NCH, _CH)
    d0 = jnp.concatenate([edge_index_md[1].astype(i32),
                          jnp.full((npad0,), _NG, i32)]).reshape(
                              _NW, _NCH, _CH)
    npad12 = _EP - 2 * _E1
    s12 = jnp.concatenate([edge_index_ml[0].astype(i32),
                           edge_index_dl[0].astype(i32),
                           jnp.zeros((npad12,), i32)]).reshape(
                               _NW, _NCH, _CH)
    d12 = jnp.concatenate([edge_index_ml[1].astype(i32),
                           edge_index_dl[1].astype(i32),
                           jnp.full((npad12,), _NG0 - 1, i32)]).reshape(
                               _NW, _NCH, _CH)
    zer = jnp.zeros((_R12, _ZW), f32)

    pp0, pp12 = _edge_kernel()(zt0, esd0, zt1, esd1, s0, d0, s12, d12, zer)

    h = _fuse_call(pp0, pp12, d_sim, m_sim, dfc_W, dfc_b, mfc_W, mfc_b,
                   hfc_W, hfc_b)

    return _pair_kernel()(h.reshape(-1), diseases.astype(i32),
                          mirnas.astype(i32))


# speedup vs baseline: 1.1186x; 1.1186x over previous
import functools

import jax
import jax.numpy as jnp
from jax import lax
from jax.experimental import pallas as pl
from jax.experimental.pallas import tpu as pltpu
from jax.experimental.pallas import tpu_sc as plsc

_NG = 878
_NG0 = 1154
_ND = 383
_NM = 495
_FEAT = 256
_H = 8
_FA = 64
_HF = _H * _FA
_ZW = _HF + 16
_OUT = 64
_B = 16384

_NC = 2
_NS = 16
_NW = _NC * _NS

_T0 = 880
_T1 = 1160
_N0P = 896
_N1S = 512
_N12 = 896
_R0 = _N0P // _NS
_R12 = _N12 // _NS

_CH = 64
_E0, _E1 = 20000, 10000
_NCH = 10
_EP = _NW * _NCH * _CH

_PW = _B // _NW


def _prep_body(x0, x1, p0, p1, q0, q1, zt0, zt1, esd0, esd1):
    hi = lax.Precision.HIGHEST
    f32 = jnp.float32

    def mm(a, b):
        return lax.dot_general(a, b, (((1,), (0,)), ((), ())),
                               precision=hi, preferred_element_type=f32)

    zt0[...] = jnp.concatenate(
        [mm(x0[...], p0[...]), jnp.zeros((_T0 - _NG, _ZW), f32)], axis=0)
    zt1[...] = jnp.concatenate(
        [mm(x1[...], p1[...]), jnp.zeros((_T1 - _NG0, _ZW), f32)], axis=0)
    e0 = mm(x0[...], q0[...])
    e1 = mm(x1[...], q1[...])
    esd0[...] = jnp.concatenate(
        [jnp.concatenate([e0, e0], axis=1),
         jnp.zeros((_T0 - _NG, 16), f32)], axis=0)
    esd1[...] = jnp.concatenate(
        [jnp.concatenate([e1, e1], axis=1),
         jnp.zeros((_T1 - _NG0, 16), f32)], axis=0)


def _prep_call(x0, x1, p0, p1, q0, q1):
    return pl.pallas_call(
        _prep_body,
        out_shape=[
            jax.ShapeDtypeStruct((_T0, _ZW), jnp.float32),
            jax.ShapeDtypeStruct((_T1, _ZW), jnp.float32),
            jax.ShapeDtypeStruct((_T0, 16), jnp.float32),
            jax.ShapeDtypeStruct((_T1, 16), jnp.float32),
        ],
    )(x0, x1, p0, p1, q0, q1)


@functools.cache
def _mesh():
    return plsc.VectorSubcoreMesh(core_axis_name="c", subcore_axis_name="s",
                                  num_cores=_NC, num_subcores=_NS)


@functools.cache
def _edge_kernel():
    return functools.partial(
        pl.kernel,
        out_type=[
            jax.ShapeDtypeStruct((_NC, _N0P, _ZW), jnp.float32),
            jax.ShapeDtypeStruct((_NC, _N12, _ZW), jnp.float32),
        ],
        mesh=_mesh(),
        compiler_params=pltpu.CompilerParams(needs_layout_passes=False,
                                             use_tc_tiling_on_sc=False),
        scratch_types=[
            pltpu.VMEM_SHARED((_N0P, _ZW), jnp.float32),
            pltpu.VMEM_SHARED((_N12, _ZW), jnp.float32),
            pltpu.VMEM((_CH, _ZW), jnp.float32),
            pltpu.VMEM((_CH, _ZW), jnp.float32),
            pltpu.VMEM((_CH, 16), jnp.float32),
            pltpu.VMEM((_CH, 16), jnp.float32),
            pltpu.VMEM((_NCH, _CH), jnp.int32),
            pltpu.VMEM((_NCH, _CH), jnp.int32),
            pltpu.SemaphoreType.DMA,
            pltpu.SemaphoreType.DMA,
            pltpu.SemaphoreType.DMA,
            pltpu.SemaphoreType.DMA,
            pltpu.SemaphoreType.DMA,
            pltpu.SemaphoreType.DMA,
        ],
    )(_edge_body)


def _edge_body(zt0, esd0, zt1, esd1, s0r, d0r, s12r, d12r, zer,
               out0, out12, acc0, acc12,
               zb0, zb1, eb0, eb1, s2d, d2d,
               gz0, gz1, ge0, ge1, sc0, sc1):
    c = lax.axis_index("c")
    s = lax.axis_index("s")
    wid = c * _NS + s
    zb = (zb0, zb1)
    eb = (eb0, eb1)
    gz = (gz0, gz1)
    ge = (ge0, ge1)
    sc = (sc0, sc1)

    pltpu.sync_copy(zer.at[pl.ds(0, _R0)], acc0.at[pl.ds(s * _R0, _R0)])
    pltpu.sync_copy(zer, acc12.at[pl.ds(s * _R12, _R12)])
    plsc.subcore_barrier()

    iota16 = lax.iota(jnp.int32, 16)
    lane_lt8 = iota16 < 8

    def remap0(v, gidx):
        return jnp.where(v < _NG, v, _NG + 2 + iota16)

    def remap12(v, gidx):
        t = v - _ND
        r1 = jnp.where((t >= 0) & (t < _NM), t, _NM + iota16)
        r2 = jnp.where(v < _ND, v + 511, _NM + iota16)
        return jnp.where(gidx < _E1, r1, r2)

    def do_graph(zt, esd, sarr, darr, acc, remap):
        pltpu.sync_copy(sarr.at[wid], s2d)
        pltpu.sync_copy(darr.at[wid], d2d)

        def issue(j, b):
            pltpu.async_copy(zt.at[s2d.at[j]], zb[b], gz[b])
            pltpu.async_copy(esd.at[d2d.at[j]], eb[b], ge[b])

        def wait_gather(j, b):
            pltpu.make_async_copy(zt.at[s2d.at[j]], zb[b], gz[b]).wait()
            pltpu.make_async_copy(esd.at[d2d.at[j]], eb[b], ge[b]).wait()

        def wait_scatter(j, b):
            pltpu.make_async_copy(zb[b], acc.at[d2d.at[j]], sc[b]).wait()

        issue(0, 0)

        def step(j, b):
            zrows = zb[b]
            edrows = eb[b]

            @pl.when(j >= 1)
            def _():
                wait_scatter(j - 1, 1 - b)

            @pl.when(j < _NCH - 1)
            def _():
                issue(j + 1, 1 - b)

            wait_gather(j, b)

            base = wid * (_NCH * _CH) + j * _CH
            for q in range(_CH // 16):
                sl = pl.ds(q * 16, 16)
                gidx = base + q * 16 + iota16
                d2d[j, sl] = remap(d2d[j, sl], gidx)

            def edge(e, carry2):
                esv = zrows[e, pl.ds(_HF, 16)]
                edv = edrows[e, :]
                x = esv + edv
                lk = jnp.where(x >= 0, x, jnp.float32(0.2) * x)
                w = jnp.exp(lk)
                wm = jnp.where(lane_lt8, w, jnp.float32(0.0))
                dn = lax.GatherDimensionNumbers(
                    offset_dims=(), collapsed_slice_dims=(0,),
                    start_index_map=(0,))
                for h in range(_H):
                    wspl = lax.gather(
                        w, jnp.full((16, 1), h, jnp.int32), dn, (1,),
                        mode=lax.GatherScatterMode.PROMISE_IN_BOUNDS)
                    for q in range(4):
                        t = h * 4 + q
                        sl = pl.ds(t * 16, 16)
                        zrows[e, sl] = zrows[e, sl] * wspl
                zrows[e, pl.ds(_HF, 16)] = wm
                return carry2

            lax.fori_loop(0, _CH, edge, 0)
            pltpu.async_copy(zrows, acc.at[d2d.at[j]], sc[b], add=True)

        def pair(j2, carry):
            step(j2 * 2, 0)
            step(j2 * 2 + 1, 1)
            return carry

        lax.fori_loop(0, _NCH // 2, pair, 0)
        wait_scatter(_NCH - 1, 1)

    do_graph(zt0, esd0, s0r, d0r, acc0, remap0)
    do_graph(zt1, esd1, s12r, d12r, acc12, remap12)
    plsc.subcore_barrier()

    pltpu.sync_copy(acc0.at[pl.ds(s * _R0, _R0)],
                    out0.at[c, pl.ds(s * _R0, _R0)])
    pltpu.sync_copy(acc12.at[pl.ds(s * _R12, _R12)],
                    out12.at[c, pl.ds(s * _R12, _R12)])


def _fuse_body(p0, p12, dsim, msim, dfcw, dfcb,
               mfcw, mfcb, hfcw, hfcb, h_ref):
    hi = lax.Precision.HIGHEST
    f32 = jnp.float32

    def mm(a, b):
        return lax.dot_general(a, b, (((1,), (0,)), ((), ())),
                               precision=hi, preferred_element_type=f32)

    def elu(x):
        return jnp.where(x > 0, x, jnp.exp(jnp.minimum(x, 0.0)) - 1.0)

    ii = lax.broadcasted_iota(jnp.int32, (_H, _HF), 0)
    jj = lax.broadcasted_iota(jnp.int32, (_H, _HF), 1)
    r8 = (jj // _FA == ii).astype(f32)

    def hpart(p):
        a = p[0] + p[1]
        msg = a[:, :_HF]
        den = a[:, _HF:_HF + _H]
        dinv = 1.0 / (den + 1e-9)
        return elu(msg * mm(dinv, r8))

    h0 = hpart(p0[...])
    h12 = hpart(p12[...])
    hs1 = 0.5 * (h0[:_ND + 1] + h12[511:511 + _ND + 1])
    hs2 = 0.5 * (h0[_ND:_ND + _NM + 1] + h12[:_NM + 1])
    dss = jnp.concatenate(
        [dsim[:_ND], jnp.zeros((1, _ND), f32)], axis=0)
    mss = jnp.concatenate(
        [msim[_ND:_NG], jnp.zeros((1, _NM), f32)], axis=0)
    dw = dfcw[...]
    mw = mfcw[...]
    hd = elu(mm(hs1, dw[:_HF]) + mm(dss, dw[_HF:])
             + dfcb[...].reshape(1, _OUT))
    hm = elu(mm(hs2, mw[:_HF]) + mm(mss, mw[_HF:])
             + mfcb[...].reshape(1, _OUT))
    hcat = jnp.concatenate([hd[:_ND], hm[:_NM]], axis=0)
    h = elu(mm(hcat, hfcw[...]) + hfcb[...].reshape(1, _OUT))
    h_ref[...] = jnp.transpose(
        jnp.concatenate([h, jnp.zeros((_N0P - _NG, _OUT), f32)], axis=0),
        (1, 0))


def _fuse_call(p0, p12, dsim, msim, dfcw, dfcb, mfcw, mfcb, hfcw, hfcb):
    return pl.pallas_call(
        _fuse_body,
        out_shape=jax.ShapeDtypeStruct((_OUT, _N0P), jnp.float32),
    )(p0, p12, dsim, msim, dfcw, dfcb, mfcw, mfcb, hfcw, hfcb)


@functools.cache
def _pair_kernel():
    return functools.partial(
        pl.kernel,
        out_type=jax.ShapeDtypeStruct((_B,), jnp.float32),
        mesh=_mesh(),
        compiler_params=pltpu.CompilerParams(needs_layout_passes=False, use_tc_tiling_on_sc=False),
        scratch_types=[
            pltpu.VMEM((_N0P * _OUT,), jnp.float32),
            pltpu.VMEM((_PW,), jnp.int32),
            pltpu.VMEM((_PW,), jnp.int32),
            pltpu.VMEM((_PW,), jnp.float32),
        ],
    )(_pair_body)


def _pair_body(hflat, dis, mir, out, hbuf, dbuf, mbuf, obuf):
    c = lax.axis_index("c")
    s = lax.axis_index("s")
    wid = c * _NS + s
    pltpu.sync_copy(hflat, hbuf)
    pltpu.sync_copy(dis.at[pl.ds(wid * _PW, _PW)], dbuf)
    pltpu.sync_copy(mir.at[pl.ds(wid * _PW, _PW)], mbuf)

    def grp(g, carry):
        dv = dbuf[pl.ds(g * 16, 16)]
        mv = mbuf[pl.ds(g * 16, 16)]
        accs = [jnp.zeros((16,), jnp.float32) for _ in range(4)]
        for k in range(_OUT):
            a = plsc.load_gather(hbuf, [dv + k * _N0P])
            b = plsc.load_gather(hbuf, [mv + k * _N0P])
            accs[k % 4] = accs[k % 4] + a * b
        acc = (accs[0] + accs[1]) + (accs[2] + accs[3])
        obuf[pl.ds(g * 16, 16)] = 1.0 / (1.0 + jnp.exp(-acc))
        return carry

    lax.fori_loop(0, _PW // 16, grp, 0)
    pltpu.sync_copy(obuf, out.at[pl.ds(wid * _PW, _PW)])


def kernel(x_g, x_g0, d_sim, m_sim, edge_index_md, edge_index_ml,
           edge_index_dl, diseases, mirnas, gat_W, gat_asrc, gat_adst,
           mp_W, mp_asrc, mp_adst, sem_P1, sem_b1, sem_P2, mfc_W, mfc_b,
           dfc_W, dfc_b, hfc_W, hfc_b):
    f32 = jnp.float32
    i32 = jnp.int32

    wf0 = gat_W.reshape(_FEAT, _HF)
    was0 = jnp.einsum('fhk,hk->fh', gat_W, gat_asrc)
    wad0 = jnp.einsum('fhk,hk->fh', gat_W, gat_adst)
    p0w = jnp.concatenate([wf0, was0, wad0], axis=1)
    wf1 = jnp.transpose(mp_W, (1, 0, 2)).reshape(_FEAT, _HF)
    was1 = jnp.einsum('hfk,hk->fh', mp_W, mp_asrc)
    wad1 = jnp.einsum('hfk,hk->fh', mp_W, mp_adst)
    p1w = jnp.concatenate([wf1, was1, wad1], axis=1)

    zt0, zt1, esd0, esd1 = _prep_call(x_g, x_g0, p0w, p1w, wad0, wad1)

    npad0 = _EP - _E0
    s0 = jnp.concatenate([edge_index_md[0].astype(i32),
                          jnp.zeros((npad0,), i32)]).reshape(_NW, _NCH, _CH)
    d0 = jnp.concatenate([edge_index_md[1].astype(i32),
                          jnp.full((npad0,), _NG, i32)]).reshape(
                              _NW, _NCH, _CH)
    npad12 = _EP - 2 * _E1
    s12 = jnp.concatenate([edge_index_ml[0].astype(i32),
                           edge_index_dl[0].astype(i32),
                           jnp.zeros((npad12,), i32)]).reshape(
                               _NW, _NCH, _CH)
    d12 = jnp.concatenate([edge_index_ml[1].astype(i32),
                           edge_index_dl[1].astype(i32),
                           jnp.full((npad12,), _NG0 - 1, i32)]).reshape(
                               _NW, _NCH, _CH)
    zer = jnp.zeros((_R12, _ZW), f32)

    pp0, pp12 = _edge_kernel()(zt0, esd0, zt1, esd1, s0, d0, s12, d12, zer)

    h = _fuse_call(pp0, pp12, d_sim, m_sim, dfc_W, dfc_b, mfc_W, mfc_b,
                   hfc_W, hfc_b)

    return _pair_kernel()(h.reshape(-1), diseases.astype(i32),
                          mirnas.astype(i32))
